# baseline (speedup 1.0000x reference)
import jax
import jax.numpy as jnp
from jax import lax
from jax.experimental import pallas as pl
from jax.experimental.pallas import tpu as pltpu

_KB = 64


def _encoder_kernel(wt_ref, tt_ref, bt_ref, rsel_ref, out_ref,
                    cur_ref, bl_ref, bv_ref):
    s = pl.program_id(0)
    kb = pl.program_id(1)
    nkb = pl.num_programs(1)
    d = tt_ref.shape[0]
    n = tt_ref.shape[1]

    @pl.when(jnp.logical_and(s == 0, kb == 0))
    def _init_current():
        cur_ref[...] = jnp.zeros((d, n), jnp.bfloat16)

    @pl.when(kb == 0)
    def _init_best():
        bl_ref[...] = jnp.full((1, n), jnp.inf, jnp.float32)

    outs = jnp.dot(wt_ref[0], cur_ref[...],
                   preferred_element_type=jnp.float32)
    outs = outs.astype(jnp.bfloat16) + bt_ref[0]
    outs3 = outs.reshape(_KB, d, n)

    diff = outs3 - tt_ref[...][None, :, :]
    sq = (diff * diff).reshape(_KB * d, n)
    losses = jnp.dot(rsel_ref[...], sq, preferred_element_type=jnp.float32)

    bmin = jnp.min(losses, axis=0)
    kiota = lax.broadcasted_iota(jnp.int32, (_KB, n), 0)
    bidx = jnp.min(jnp.where(losses <= bmin[None, :], kiota, _KB), axis=0)
    onehot = (kiota == bidx[None, :]).astype(jnp.bfloat16)
    bvec = jnp.sum(outs3 * onehot[:, None, :], axis=0)

    prev = bl_ref[...]
    better = bmin[None, :] < prev
    bl_ref[...] = jnp.where(better, bmin[None, :], prev)
    bv_ref[...] = jnp.where(better, bvec, bv_ref[...])

    @pl.when(kb == nkb - 1)
    def _finish_stage():
        cur_ref[...] = bv_ref[...]
        out_ref[0] = bv_ref[...].astype(jnp.float32)


def kernel(targets, W, b):
    num_stages, psize, kd = W.shape
    batch = targets.shape[0]
    nkb = (kd // psize) // _KB
    kbs = _KB * psize

    wt = W.transpose(0, 2, 1).astype(jnp.bfloat16)
    tt = targets.T.astype(jnp.bfloat16)
    bt = b.reshape(num_stages, kd, 1).astype(jnp.bfloat16)
    rsel = (jnp.arange(kbs, dtype=jnp.int32)[None, :] // psize
            == jnp.arange(_KB, dtype=jnp.int32)[:, None]).astype(jnp.bfloat16)

    out_t = pl.pallas_call(
        _encoder_kernel,
        grid=(num_stages, nkb),
        in_specs=[
            pl.BlockSpec((1, kbs, psize), lambda s, kb: (s, kb, 0)),
            pl.BlockSpec((psize, batch), lambda s, kb: (0, 0)),
            pl.BlockSpec((1, kbs, 1), lambda s, kb: (s, kb, 0)),
            pl.BlockSpec((_KB, kbs), lambda s, kb: (0, 0)),
        ],
        out_specs=pl.BlockSpec((1, psize, batch), lambda s, kb: (s, 0, 0)),
        out_shape=jax.ShapeDtypeStruct((num_stages, psize, batch), jnp.float32),
        scratch_shapes=[
            pltpu.VMEM((psize, batch), jnp.bfloat16),
            pltpu.VMEM((1, batch), jnp.float32),
            pltpu.VMEM((psize, batch), jnp.bfloat16),
        ],
        compiler_params=pltpu.CompilerParams(
            dimension_semantics=("arbitrary", "arbitrary"),
        ),
    )(wt, tt, bt, rsel)

    return out_t.transpose(2, 0, 1)


# speedup vs baseline: 1.5188x; 1.5188x over previous
import jax
import jax.numpy as jnp
from jax import lax
from jax.experimental import pallas as pl
from jax.experimental.pallas import tpu as pltpu

_KB = 64


def _encoder_kernel(w_ref, tt_ref, b_ref, rsel_ref, out_ref,
                    cur_ref, bl_ref, bv_ref):
    s = pl.program_id(0)
    kb = pl.program_id(1)
    nkb = pl.num_programs(1)
    d = tt_ref.shape[0]
    n = tt_ref.shape[1]

    @pl.when(jnp.logical_and(s == 0, kb == 0))
    def _init_current():
        cur_ref[...] = jnp.zeros((d, n), jnp.bfloat16)

    @pl.when(kb == 0)
    def _init_best():
        bl_ref[...] = jnp.full((1, n), jnp.inf, jnp.float32)

    w_bf = w_ref[0].astype(jnp.bfloat16)
    outs = lax.dot_general(w_bf, cur_ref[...],
                           ((( 0,), (0,)), ((), ())),
                           preferred_element_type=jnp.float32)
    b_col = jnp.swapaxes(b_ref[0], 0, 1)
    outs = (outs + b_col).astype(jnp.bfloat16)
    outs3 = outs.reshape(_KB, d, n)

    diff = outs3 - tt_ref[...][None, :, :]
    sq = (diff * diff).reshape(_KB * d, n)
    losses = jnp.dot(rsel_ref[...], sq, preferred_element_type=jnp.float32)

    bmin = jnp.min(losses, axis=0)
    kiota = lax.broadcasted_iota(jnp.int32, (_KB, n), 0)
    bidx = jnp.min(jnp.where(losses <= bmin[None, :], kiota, _KB), axis=0)
    onehot = (kiota == bidx[None, :]).astype(jnp.bfloat16)
    bvec = jnp.sum(outs3 * onehot[:, None, :], axis=0,
                   dtype=jnp.bfloat16)

    prev = bl_ref[...]
    better = bmin[None, :] < prev
    bl_ref[...] = jnp.where(better, bmin[None, :], prev)
    bv_ref[...] = jnp.where(better, bvec, bv_ref[...])

    @pl.when(kb == nkb - 1)
    def _finish_stage():
        cur_ref[...] = bv_ref[...]
        out_ref[0] = jnp.swapaxes(bv_ref[...], 0, 1).astype(jnp.float32)


def kernel(targets, W, b):
    num_stages, psize, kd = W.shape
    batch = targets.shape[0]
    nkb = (kd // psize) // _KB
    kbs = _KB * psize

    tt = targets.T.astype(jnp.bfloat16)
    b3 = b.reshape(num_stages, 1, kd)
    rsel = (jnp.arange(kbs, dtype=jnp.int32)[None, :] // psize
            == jnp.arange(_KB, dtype=jnp.int32)[:, None]).astype(jnp.bfloat16)

    out = pl.pallas_call(
        _encoder_kernel,
        grid=(num_stages, nkb),
        in_specs=[
            pl.BlockSpec((1, psize, kbs), lambda s, kb: (s, 0, kb)),
            pl.BlockSpec((psize, batch), lambda s, kb: (0, 0)),
            pl.BlockSpec((1, 1, kbs), lambda s, kb: (s, 0, kb)),
            pl.BlockSpec((_KB, kbs), lambda s, kb: (0, 0)),
        ],
        out_specs=pl.BlockSpec((1, batch, psize), lambda s, kb: (s, 0, 0)),
        out_shape=jax.ShapeDtypeStruct((num_stages, batch, psize), jnp.float32),
        scratch_shapes=[
            pltpu.VMEM((psize, batch), jnp.bfloat16),
            pltpu.VMEM((1, batch), jnp.float32),
            pltpu.VMEM((psize, batch), jnp.bfloat16),
        ],
        compiler_params=pltpu.CompilerParams(
            dimension_semantics=("arbitrary", "arbitrary"),
        ),
    )(W, tt, b3, rsel)

    return out.transpose(1, 0, 2)
